# baseline (speedup 1.0000x reference)
import jax
import jax.numpy as jnp
from jax import lax
from jax.experimental import pallas as pl
from jax.experimental.pallas import tpu as pltpu
from jax.experimental.pallas import tpu_sc as plsc

EMBED = 32
BATCH = 16384
NUM_ROWS = 1000000
CHUNK = 250112
NBLK = CHUNK // 128
COL_BLOCKS = (NUM_ROWS + 127) // 128 - 1
NUM_WORKERS = 32
B_PER_W = BATCH // NUM_WORKERS


def _repack_body(i0, i1, i2, i3, out_ref):
    for u, ref in enumerate((i0, i1, i2, i3)):
        out_ref[:, 32 * u:32 * u + 32] = jnp.transpose(ref[...])


def _repack(tabT):
    specs = [
        pl.BlockSpec((32, 128),
                     (lambda k, uu=u: (0, jnp.minimum(NBLK * uu + k, COL_BLOCKS))))
        for u in range(4)
    ]
    return pl.pallas_call(
        _repack_body,
        grid=(NBLK,),
        in_specs=specs,
        out_specs=pl.BlockSpec((128, 128), lambda k: (k, 0)),
        out_shape=jax.ShapeDtypeStruct((CHUNK, 128), jnp.float32),
    )(tabT, tabT, tabT, tabT)


def _gather_body(uidx_hbm, midx_hbm, uR_hbm, mR_hbm, ug_hbm, mg_hbm,
                 idx_v, q_v, rows_v, sem):
    wid = lax.axis_index("s") * 2 + lax.axis_index("c")
    base = wid * B_PER_W
    for idx_hbm, R_hbm, out_hbm in ((uidx_hbm, uR_hbm, ug_hbm),
                                    (midx_hbm, mR_hbm, mg_hbm)):
        pltpu.sync_copy(idx_hbm.at[pl.ds(base, B_PER_W)], idx_v)
        for j in range(B_PER_W // 16):
            v = idx_v[pl.ds(16 * j, 16)]
            q_v[pl.ds(16 * j, 16)] = lax.rem(v, CHUNK)
        pltpu.async_copy(R_hbm.at[q_v], rows_v, sem).wait()
        pltpu.sync_copy(rows_v, out_hbm.at[pl.ds(base, B_PER_W)])


def _sc_gather(uidx, midx, uR, mR):
    mesh = plsc.VectorSubcoreMesh(core_axis_name="c", subcore_axis_name="s")
    return pl.kernel(
        _gather_body,
        mesh=mesh,
        out_type=[
            jax.ShapeDtypeStruct((BATCH, 128), jnp.float32),
            jax.ShapeDtypeStruct((BATCH, 128), jnp.float32),
        ],
        scratch_types=[
            pltpu.VMEM((B_PER_W,), jnp.int32),
            pltpu.VMEM((B_PER_W,), jnp.int32),
            pltpu.VMEM((B_PER_W, 128), jnp.float32),
            pltpu.SemaphoreType.DMA,
        ],
    )(uidx, midx, uR, mR)


def _mlp_body(ug_ref, mg_ref, ui_ref, mi_ref, w1a_ref, w1b_ref, b1_ref,
              w2_ref, b2_ref, o_ref):
    ou = ui_ref[...] // CHUNK
    om = mi_ref[...] // CHUNK
    wu = ug_ref[...]
    wm = mg_ref[...]
    h = b1_ref[...]
    for t in range(4):
        hu = jnp.dot(wu[:, 32 * t:32 * t + 32], w1a_ref[...],
                     preferred_element_type=jnp.float32)
        hm = jnp.dot(wm[:, 32 * t:32 * t + 32], w1b_ref[...],
                     preferred_element_type=jnp.float32)
        h = h + jnp.where(ou == t, hu, 0.0) + jnp.where(om == t, hm, 0.0)
    h = jnp.maximum(h, 0.0)
    o = jnp.dot(h, w2_ref[...], preferred_element_type=jnp.float32) + b2_ref[...]
    o_ref[...] = 5.0 * jax.nn.sigmoid(o)


def kernel(inputs, user_embedding, movie_embedding, W1, b1, W2, b2):
    uidx = inputs[:, 0]
    midx = inputs[:, 1]

    uR = _repack(user_embedding.T)
    mR = _repack(movie_embedding.T)
    ug, mg = _sc_gather(uidx, midx, uR, mR)

    BT = 2048
    out = pl.pallas_call(
        _mlp_body,
        grid=(BATCH // BT,),
        in_specs=[
            pl.BlockSpec((BT, 128), lambda i: (i, 0)),
            pl.BlockSpec((BT, 128), lambda i: (i, 0)),
            pl.BlockSpec((BT, 1), lambda i: (i, 0)),
            pl.BlockSpec((BT, 1), lambda i: (i, 0)),
            pl.BlockSpec((EMBED, EMBED), lambda i: (0, 0)),
            pl.BlockSpec((EMBED, EMBED), lambda i: (0, 0)),
            pl.BlockSpec((1, EMBED), lambda i: (0, 0)),
            pl.BlockSpec((EMBED, 1), lambda i: (0, 0)),
            pl.BlockSpec((1, 1), lambda i: (0, 0)),
        ],
        out_specs=pl.BlockSpec((BT, 1), lambda i: (i, 0)),
        out_shape=jax.ShapeDtypeStruct((BATCH, 1), jnp.float32),
    )(ug, mg, uidx.reshape(-1, 1), midx.reshape(-1, 1),
      W1[:EMBED], W1[EMBED:], b1.reshape(1, EMBED), W2, b2.reshape(1, 1))
    return out.reshape(-1)


# speedup vs baseline: 4.0629x; 4.0629x over previous
import jax
import jax.numpy as jnp
from jax import lax
from jax.experimental import pallas as pl
from jax.experimental.pallas import tpu as pltpu
from jax.experimental.pallas import tpu_sc as plsc

EMBED = 32
BATCH = 16384
NUM_ROWS = 1000000
CHUNK = 1 << 18
OUT_BLK = 2048
GRID = CHUNK // OUT_BLK
COL_BLOCKS = (NUM_ROWS + OUT_BLK - 1) // OUT_BLK - 1
NUM_WORKERS = 32
B_PER_W = BATCH // NUM_WORKERS


def _repack_body(i0, i1, i2, i3, eye_ref, out_ref):
    ey = eye_ref[...]
    for u, ref in enumerate((i0, i1, i2, i3)):
        out_ref[:, 32 * u:32 * u + 32] = lax.dot_general(
            ref[...], ey, (((0,), (0,)), ((), ())),
            preferred_element_type=jnp.float32)


def _repack(tabT, eye):
    specs = [
        pl.BlockSpec((32, OUT_BLK),
                     (lambda k, uu=u: (0, jnp.minimum(GRID * uu + k, COL_BLOCKS))))
        for u in range(4)
    ]
    specs.append(pl.BlockSpec((EMBED, EMBED), lambda k: (0, 0)))
    return pl.pallas_call(
        _repack_body,
        grid=(GRID,),
        in_specs=specs,
        out_specs=pl.BlockSpec((OUT_BLK, 128), lambda k: (k, 0)),
        out_shape=jax.ShapeDtypeStruct((CHUNK, 128), jnp.float32),
        compiler_params=pltpu.CompilerParams(fuse_transposed_lhs_in_matmul=True),
    )(tabT, tabT, tabT, tabT, eye)


def _gather_body(uidx_hbm, midx_hbm, uR_hbm, mR_hbm, ug_hbm, mg_hbm,
                 idx_v, q_v, rows_v, sem):
    wid = lax.axis_index("s") * 2 + lax.axis_index("c")
    base = wid * B_PER_W
    for idx_hbm, R_hbm, out_hbm in ((uidx_hbm, uR_hbm, ug_hbm),
                                    (midx_hbm, mR_hbm, mg_hbm)):
        pltpu.sync_copy(idx_hbm.at[pl.ds(base, B_PER_W)], idx_v)
        for j in range(B_PER_W // 16):
            v = idx_v[pl.ds(16 * j, 16)]
            q_v[pl.ds(16 * j, 16)] = jnp.bitwise_and(v, CHUNK - 1)
        pltpu.async_copy(R_hbm.at[q_v], rows_v, sem).wait()
        pltpu.sync_copy(rows_v, out_hbm.at[pl.ds(base, B_PER_W)])


def _sc_gather(uidx, midx, uR, mR):
    mesh = plsc.VectorSubcoreMesh(core_axis_name="c", subcore_axis_name="s")
    return pl.kernel(
        _gather_body,
        mesh=mesh,
        out_type=[
            jax.ShapeDtypeStruct((BATCH, 128), jnp.float32),
            jax.ShapeDtypeStruct((BATCH, 128), jnp.float32),
        ],
        scratch_types=[
            pltpu.VMEM((B_PER_W,), jnp.int32),
            pltpu.VMEM((B_PER_W,), jnp.int32),
            pltpu.VMEM((B_PER_W, 128), jnp.float32),
            pltpu.SemaphoreType.DMA,
        ],
    )(uidx, midx, uR, mR)


def _mlp_body(ug_ref, mg_ref, uoh_ref, moh_ref, e_ref, w1a_ref, w1b_ref,
              b1_ref, w2_ref, b2_ref, o_ref):
    mu = jnp.dot(uoh_ref[...], e_ref[...], preferred_element_type=jnp.float32)
    mm = jnp.dot(moh_ref[...], e_ref[...], preferred_element_type=jnp.float32)
    h = (jnp.dot(ug_ref[...] * mu, w1a_ref[...],
                 preferred_element_type=jnp.float32)
         + jnp.dot(mg_ref[...] * mm, w1b_ref[...],
                   preferred_element_type=jnp.float32)
         + b1_ref[...])
    h = jnp.maximum(h, 0.0)
    o = jnp.dot(h, w2_ref[...], preferred_element_type=jnp.float32) + b2_ref[...]
    o_ref[...] = 5.0 * jax.nn.sigmoid(o)


def kernel(inputs, user_embedding, movie_embedding, W1, b1, W2, b2):
    uidx = inputs[:, 0]
    midx = inputs[:, 1]
    eye = jnp.eye(EMBED, dtype=jnp.float32)

    uR = _repack(user_embedding.T, eye)
    mR = _repack(movie_embedding.T, eye)
    ug, mg = _sc_gather(uidx, midx, uR, mR)

    uoh = jax.nn.one_hot(uidx // CHUNK, 4, dtype=jnp.float32)
    moh = jax.nn.one_hot(midx // CHUNK, 4, dtype=jnp.float32)
    expander = jnp.repeat(jnp.eye(4, dtype=jnp.float32), EMBED, axis=1)
    w1a_stack = jnp.tile(W1[:EMBED], (4, 1))
    w1b_stack = jnp.tile(W1[EMBED:], (4, 1))

    BT = 4096
    out = pl.pallas_call(
        _mlp_body,
        grid=(BATCH // BT,),
        in_specs=[
            pl.BlockSpec((BT, 128), lambda i: (i, 0)),
            pl.BlockSpec((BT, 128), lambda i: (i, 0)),
            pl.BlockSpec((BT, 4), lambda i: (i, 0)),
            pl.BlockSpec((BT, 4), lambda i: (i, 0)),
            pl.BlockSpec((4, 128), lambda i: (0, 0)),
            pl.BlockSpec((128, EMBED), lambda i: (0, 0)),
            pl.BlockSpec((128, EMBED), lambda i: (0, 0)),
            pl.BlockSpec((1, EMBED), lambda i: (0, 0)),
            pl.BlockSpec((EMBED, 1), lambda i: (0, 0)),
            pl.BlockSpec((1, 1), lambda i: (0, 0)),
        ],
        out_specs=pl.BlockSpec((BT, 1), lambda i: (i, 0)),
        out_shape=jax.ShapeDtypeStruct((BATCH, 1), jnp.float32),
    )(ug, mg, uoh, moh, expander, w1a_stack, w1b_stack,
      b1.reshape(1, EMBED), W2, b2.reshape(1, 1))
    return out.reshape(-1)


# speedup vs baseline: 6.9797x; 1.7179x over previous
import jax
import jax.numpy as jnp
from jax import lax
from jax.experimental import pallas as pl
from jax.experimental.pallas import tpu as pltpu
from jax.experimental.pallas import tpu_sc as plsc

EMBED = 32
BATCH = 16384
NUM_ROWS = 1000000
CHUNK = 1 << 18
OUT_BLK = 2048
GRID = CHUNK // OUT_BLK
COL_BLOCKS = (NUM_ROWS + OUT_BLK - 1) // OUT_BLK - 1
NUM_WORKERS = 32
B_PER_W = BATCH // NUM_WORKERS


def _repack_body(i0, i1, i2, i3, out_ref):
    x = jnp.concatenate([i0[...], i1[...], i2[...], i3[...]], axis=0)
    out_ref[...] = jnp.transpose(x)


def _repack(tabT):
    specs = [
        pl.BlockSpec((32, OUT_BLK),
                     (lambda k, uu=u: (0, jnp.minimum(GRID * uu + k, COL_BLOCKS))))
        for u in range(4)
    ]
    return pl.pallas_call(
        _repack_body,
        grid=(GRID,),
        in_specs=specs,
        out_specs=pl.BlockSpec((OUT_BLK, 128), lambda k: (k, 0)),
        out_shape=jax.ShapeDtypeStruct((CHUNK, 128), jnp.float32),
    )(tabT, tabT, tabT, tabT)


def _gather_body(uidx_hbm, midx_hbm, uR_hbm, mR_hbm, ug_hbm, mg_hbm,
                 idx_v, q_v, rows_v, sem):
    wid = lax.axis_index("s") * 2 + lax.axis_index("c")
    base = wid * B_PER_W
    for idx_hbm, R_hbm, out_hbm in ((uidx_hbm, uR_hbm, ug_hbm),
                                    (midx_hbm, mR_hbm, mg_hbm)):
        pltpu.sync_copy(idx_hbm.at[pl.ds(base, B_PER_W)], idx_v)
        for j in range(B_PER_W // 16):
            v = idx_v[pl.ds(16 * j, 16)]
            q_v[pl.ds(16 * j, 16)] = jnp.bitwise_and(v, CHUNK - 1)
        pltpu.async_copy(R_hbm.at[q_v], rows_v, sem).wait()
        pltpu.sync_copy(rows_v, out_hbm.at[pl.ds(base, B_PER_W)])


def _sc_gather(uidx, midx, uR, mR):
    mesh = plsc.VectorSubcoreMesh(core_axis_name="c", subcore_axis_name="s")
    return pl.kernel(
        _gather_body,
        mesh=mesh,
        out_type=[
            jax.ShapeDtypeStruct((BATCH, 128), jnp.float32),
            jax.ShapeDtypeStruct((BATCH, 128), jnp.float32),
        ],
        scratch_types=[
            pltpu.VMEM((B_PER_W,), jnp.int32),
            pltpu.VMEM((B_PER_W,), jnp.int32),
            pltpu.VMEM((B_PER_W, 128), jnp.float32),
            pltpu.SemaphoreType.DMA,
        ],
    )(uidx, midx, uR, mR)


def _mlp_body(ug_ref, mg_ref, uoh_ref, moh_ref, e_ref, w1a_ref, w1b_ref,
              b1_ref, w2_ref, b2_ref, o_ref):
    mu = jnp.dot(uoh_ref[...], e_ref[...], preferred_element_type=jnp.float32)
    mm = jnp.dot(moh_ref[...], e_ref[...], preferred_element_type=jnp.float32)
    h = (jnp.dot(ug_ref[...] * mu, w1a_ref[...],
                 preferred_element_type=jnp.float32)
         + jnp.dot(mg_ref[...] * mm, w1b_ref[...],
                   preferred_element_type=jnp.float32)
         + b1_ref[...])
    h = jnp.maximum(h, 0.0)
    o = jnp.dot(h, w2_ref[...], preferred_element_type=jnp.float32) + b2_ref[...]
    o_ref[...] = 5.0 * jax.nn.sigmoid(o)


def kernel(inputs, user_embedding, movie_embedding, W1, b1, W2, b2):
    uidx = inputs[:, 0]
    midx = inputs[:, 1]
    uR = _repack(user_embedding.T)
    mR = _repack(movie_embedding.T)
    ug, mg = _sc_gather(uidx, midx, uR, mR)

    uoh = jax.nn.one_hot(uidx // CHUNK, 4, dtype=jnp.float32)
    moh = jax.nn.one_hot(midx // CHUNK, 4, dtype=jnp.float32)
    expander = jnp.repeat(jnp.eye(4, dtype=jnp.float32), EMBED, axis=1)
    w1a_stack = jnp.tile(W1[:EMBED], (4, 1))
    w1b_stack = jnp.tile(W1[EMBED:], (4, 1))

    BT = 4096
    out = pl.pallas_call(
        _mlp_body,
        grid=(BATCH // BT,),
        in_specs=[
            pl.BlockSpec((BT, 128), lambda i: (i, 0)),
            pl.BlockSpec((BT, 128), lambda i: (i, 0)),
            pl.BlockSpec((BT, 4), lambda i: (i, 0)),
            pl.BlockSpec((BT, 4), lambda i: (i, 0)),
            pl.BlockSpec((4, 128), lambda i: (0, 0)),
            pl.BlockSpec((128, EMBED), lambda i: (0, 0)),
            pl.BlockSpec((128, EMBED), lambda i: (0, 0)),
            pl.BlockSpec((1, EMBED), lambda i: (0, 0)),
            pl.BlockSpec((EMBED, 1), lambda i: (0, 0)),
            pl.BlockSpec((1, 1), lambda i: (0, 0)),
        ],
        out_specs=pl.BlockSpec((BT, 1), lambda i: (i, 0)),
        out_shape=jax.ShapeDtypeStruct((BATCH, 1), jnp.float32),
    )(ug, mg, uoh, moh, expander, w1a_stack, w1b_stack,
      b1.reshape(1, EMBED), W2, b2.reshape(1, 1))
    return out.reshape(-1)


# speedup vs baseline: 9.1172x; 1.3063x over previous
import jax
import jax.numpy as jnp
from jax import lax
from jax.experimental import pallas as pl
from jax.experimental.pallas import tpu as pltpu
from jax.experimental.pallas import tpu_sc as plsc

EMBED = 32
BATCH = 16384
NUM_ROWS = 1000000
CHUNK = 1 << 18
OUT_BLK = 4096
GRID = CHUNK // OUT_BLK
COL_BLOCKS = (NUM_ROWS + OUT_BLK - 1) // OUT_BLK - 1
NUM_WORKERS = 32
B_PER_W = BATCH // NUM_WORKERS


def _repack_body(i0, i1, i2, i3, out_ref):
    x = jnp.concatenate([i0[...], i1[...], i2[...], i3[...]], axis=0)
    out_ref[...] = jnp.transpose(x)


def _repack(tabT):
    specs = [
        pl.BlockSpec((32, OUT_BLK),
                     (lambda k, uu=u: (0, jnp.minimum(GRID * uu + k, COL_BLOCKS))))
        for u in range(4)
    ]
    return pl.pallas_call(
        _repack_body,
        grid=(GRID,),
        in_specs=specs,
        out_specs=pl.BlockSpec((OUT_BLK, 128), lambda k: (k, 0)),
        out_shape=jax.ShapeDtypeStruct((CHUNK, 128), jnp.float32),
    )(tabT, tabT, tabT, tabT)


def _gather_body(uidx_hbm, midx_hbm, uR_hbm, mR_hbm, ug_hbm, mg_hbm,
                 idx_v, q_v, rows_v, sem):
    wid = lax.axis_index("s") * 2 + lax.axis_index("c")
    base = wid * B_PER_W
    for idx_hbm, R_hbm, out_hbm in ((uidx_hbm, uR_hbm, ug_hbm),
                                    (midx_hbm, mR_hbm, mg_hbm)):
        pltpu.sync_copy(idx_hbm.at[pl.ds(base, B_PER_W)], idx_v)
        for j in range(B_PER_W // 16):
            v = idx_v[pl.ds(16 * j, 16)]
            q_v[pl.ds(16 * j, 16)] = jnp.bitwise_and(v, CHUNK - 1)
        pltpu.async_copy(R_hbm.at[q_v], rows_v, sem).wait()
        pltpu.sync_copy(rows_v, out_hbm.at[pl.ds(base, B_PER_W)])


def _sc_gather(uidx, midx, uR, mR):
    mesh = plsc.VectorSubcoreMesh(core_axis_name="c", subcore_axis_name="s")
    return pl.kernel(
        _gather_body,
        mesh=mesh,
        out_type=[
            jax.ShapeDtypeStruct((BATCH, 128), jnp.float32),
            jax.ShapeDtypeStruct((BATCH, 128), jnp.float32),
        ],
        scratch_types=[
            pltpu.VMEM((B_PER_W,), jnp.int32),
            pltpu.VMEM((B_PER_W,), jnp.int32),
            pltpu.VMEM((B_PER_W, 128), jnp.float32),
            pltpu.SemaphoreType.DMA,
        ],
    )(uidx, midx, uR, mR)


def _mlp_body(ug_ref, mg_ref, uoh_ref, moh_ref, e_ref, w1a_ref, w1b_ref,
              b1_ref, w2_ref, b2_ref, o_ref):
    mu = jnp.dot(uoh_ref[...], e_ref[...], preferred_element_type=jnp.float32)
    mm = jnp.dot(moh_ref[...], e_ref[...], preferred_element_type=jnp.float32)
    h = (jnp.dot(ug_ref[...] * mu, w1a_ref[...],
                 preferred_element_type=jnp.float32)
         + jnp.dot(mg_ref[...] * mm, w1b_ref[...],
                   preferred_element_type=jnp.float32)
         + b1_ref[...])
    h = jnp.maximum(h, 0.0)
    o = jnp.dot(h, w2_ref[...], preferred_element_type=jnp.float32) + b2_ref[...]
    o_ref[...] = 5.0 * jax.nn.sigmoid(o)


def kernel(inputs, user_embedding, movie_embedding, W1, b1, W2, b2):
    uidx = inputs[:, 0]
    midx = inputs[:, 1]
    uR = _repack(user_embedding.T)
    mR = _repack(movie_embedding.T)
    ug, mg = _sc_gather(uidx, midx, uR, mR)

    uoh = jax.nn.one_hot(uidx // CHUNK, 4, dtype=jnp.float32)
    moh = jax.nn.one_hot(midx // CHUNK, 4, dtype=jnp.float32)
    expander = jnp.repeat(jnp.eye(4, dtype=jnp.float32), EMBED, axis=1)
    w1a_stack = jnp.tile(W1[:EMBED], (4, 1))
    w1b_stack = jnp.tile(W1[EMBED:], (4, 1))

    BT = 4096
    out = pl.pallas_call(
        _mlp_body,
        grid=(BATCH // BT,),
        in_specs=[
            pl.BlockSpec((BT, 128), lambda i: (i, 0)),
            pl.BlockSpec((BT, 128), lambda i: (i, 0)),
            pl.BlockSpec((BT, 4), lambda i: (i, 0)),
            pl.BlockSpec((BT, 4), lambda i: (i, 0)),
            pl.BlockSpec((4, 128), lambda i: (0, 0)),
            pl.BlockSpec((128, EMBED), lambda i: (0, 0)),
            pl.BlockSpec((128, EMBED), lambda i: (0, 0)),
            pl.BlockSpec((1, EMBED), lambda i: (0, 0)),
            pl.BlockSpec((EMBED, 1), lambda i: (0, 0)),
            pl.BlockSpec((1, 1), lambda i: (0, 0)),
        ],
        out_specs=pl.BlockSpec((BT, 1), lambda i: (i, 0)),
        out_shape=jax.ShapeDtypeStruct((BATCH, 1), jnp.float32),
    )(ug, mg, uoh, moh, expander, w1a_stack, w1b_stack,
      b1.reshape(1, EMBED), W2, b2.reshape(1, 1))
    return out.reshape(-1)


# speedup vs baseline: 10.3299x; 1.1330x over previous
import jax
import jax.numpy as jnp
from jax import lax
from jax.experimental import pallas as pl
from jax.experimental.pallas import tpu as pltpu
from jax.experimental.pallas import tpu_sc as plsc

EMBED = 32
BATCH = 16384
NUM_ROWS = 1000000
CHUNK = 1 << 18
OUT_BLK = 8192
GRID = CHUNK // OUT_BLK
COL_BLOCKS = (NUM_ROWS + OUT_BLK - 1) // OUT_BLK - 1
NUM_WORKERS = 32
B_PER_W = BATCH // NUM_WORKERS


def _repack_body(i0, i1, i2, i3, out_ref):
    x = jnp.concatenate([i0[...], i1[...], i2[...], i3[...]], axis=0)
    out_ref[...] = jnp.transpose(x)


def _repack(tabT):
    specs = [
        pl.BlockSpec((32, OUT_BLK),
                     (lambda k, uu=u: (0, jnp.minimum(GRID * uu + k, COL_BLOCKS))))
        for u in range(4)
    ]
    return pl.pallas_call(
        _repack_body,
        grid=(GRID,),
        in_specs=specs,
        out_specs=pl.BlockSpec((OUT_BLK, 128), lambda k: (k, 0)),
        out_shape=jax.ShapeDtypeStruct((CHUNK, 128), jnp.float32),
    )(tabT, tabT, tabT, tabT)


def _gather_body(uidx_hbm, midx_hbm, uR_hbm, mR_hbm, ug_hbm, mg_hbm,
                 idx_v, q_v, rows_v, sem):
    wid = lax.axis_index("s") * 2 + lax.axis_index("c")
    base = wid * B_PER_W
    for idx_hbm, R_hbm, out_hbm in ((uidx_hbm, uR_hbm, ug_hbm),
                                    (midx_hbm, mR_hbm, mg_hbm)):
        pltpu.sync_copy(idx_hbm.at[pl.ds(base, B_PER_W)], idx_v)
        for j in range(B_PER_W // 16):
            v = idx_v[pl.ds(16 * j, 16)]
            q_v[pl.ds(16 * j, 16)] = jnp.bitwise_and(v, CHUNK - 1)
        pltpu.async_copy(R_hbm.at[q_v], rows_v, sem).wait()
        pltpu.sync_copy(rows_v, out_hbm.at[pl.ds(base, B_PER_W)])


def _sc_gather(uidx, midx, uR, mR):
    mesh = plsc.VectorSubcoreMesh(core_axis_name="c", subcore_axis_name="s")
    return pl.kernel(
        _gather_body,
        mesh=mesh,
        out_type=[
            jax.ShapeDtypeStruct((BATCH, 128), jnp.float32),
            jax.ShapeDtypeStruct((BATCH, 128), jnp.float32),
        ],
        scratch_types=[
            pltpu.VMEM((B_PER_W,), jnp.int32),
            pltpu.VMEM((B_PER_W,), jnp.int32),
            pltpu.VMEM((B_PER_W, 128), jnp.float32),
            pltpu.SemaphoreType.DMA,
        ],
    )(uidx, midx, uR, mR)


def _mlp_body(ug_ref, mg_ref, uoh_ref, moh_ref, e_ref, w1a_ref, w1b_ref,
              b1_ref, w2_ref, b2_ref, o_ref):
    mu = jnp.dot(uoh_ref[...], e_ref[...], preferred_element_type=jnp.float32)
    mm = jnp.dot(moh_ref[...], e_ref[...], preferred_element_type=jnp.float32)
    h = (jnp.dot(ug_ref[...] * mu, w1a_ref[...],
                 preferred_element_type=jnp.float32)
         + jnp.dot(mg_ref[...] * mm, w1b_ref[...],
                   preferred_element_type=jnp.float32)
         + b1_ref[...])
    h = jnp.maximum(h, 0.0)
    o = jnp.dot(h, w2_ref[...], preferred_element_type=jnp.float32) + b2_ref[...]
    o_ref[...] = 5.0 * jax.nn.sigmoid(o)


def kernel(inputs, user_embedding, movie_embedding, W1, b1, W2, b2):
    uidx = inputs[:, 0]
    midx = inputs[:, 1]
    uR = _repack(user_embedding.T)
    mR = _repack(movie_embedding.T)
    ug, mg = _sc_gather(uidx, midx, uR, mR)

    uoh = jax.nn.one_hot(uidx // CHUNK, 4, dtype=jnp.float32)
    moh = jax.nn.one_hot(midx // CHUNK, 4, dtype=jnp.float32)
    expander = jnp.repeat(jnp.eye(4, dtype=jnp.float32), EMBED, axis=1)
    w1a_stack = jnp.tile(W1[:EMBED], (4, 1))
    w1b_stack = jnp.tile(W1[EMBED:], (4, 1))

    BT = 4096
    out = pl.pallas_call(
        _mlp_body,
        grid=(BATCH // BT,),
        in_specs=[
            pl.BlockSpec((BT, 128), lambda i: (i, 0)),
            pl.BlockSpec((BT, 128), lambda i: (i, 0)),
            pl.BlockSpec((BT, 4), lambda i: (i, 0)),
            pl.BlockSpec((BT, 4), lambda i: (i, 0)),
            pl.BlockSpec((4, 128), lambda i: (0, 0)),
            pl.BlockSpec((128, EMBED), lambda i: (0, 0)),
            pl.BlockSpec((128, EMBED), lambda i: (0, 0)),
            pl.BlockSpec((1, EMBED), lambda i: (0, 0)),
            pl.BlockSpec((EMBED, 1), lambda i: (0, 0)),
            pl.BlockSpec((1, 1), lambda i: (0, 0)),
        ],
        out_specs=pl.BlockSpec((BT, 1), lambda i: (i, 0)),
        out_shape=jax.ShapeDtypeStruct((BATCH, 1), jnp.float32),
    )(ug, mg, uoh, moh, expander, w1a_stack, w1b_stack,
      b1.reshape(1, EMBED), W2, b2.reshape(1, 1))
    return out.reshape(-1)


# speedup vs baseline: 10.5448x; 1.0208x over previous
import jax
import jax.numpy as jnp
from jax import lax
from jax.experimental import pallas as pl
from jax.experimental.pallas import tpu as pltpu
from jax.experimental.pallas import tpu_sc as plsc

EMBED = 32
BATCH = 16384
NUM_ROWS = 1000000
CHUNK = 1 << 18
OUT_BLK = 16384
GRID = CHUNK // OUT_BLK
COL_BLOCKS = (NUM_ROWS + OUT_BLK - 1) // OUT_BLK - 1
NUM_WORKERS = 32
B_PER_W = BATCH // NUM_WORKERS


def _repack_body(i0, i1, i2, i3, out_ref):
    x = jnp.concatenate([i0[...], i1[...], i2[...], i3[...]], axis=0)
    out_ref[...] = jnp.transpose(x)


def _repack(tabT):
    specs = [
        pl.BlockSpec((32, OUT_BLK),
                     (lambda k, uu=u: (0, jnp.minimum(GRID * uu + k, COL_BLOCKS))))
        for u in range(4)
    ]
    return pl.pallas_call(
        _repack_body,
        grid=(GRID,),
        in_specs=specs,
        out_specs=pl.BlockSpec((OUT_BLK, 128), lambda k: (k, 0)),
        out_shape=jax.ShapeDtypeStruct((CHUNK, 128), jnp.float32),
    )(tabT, tabT, tabT, tabT)


def _gather_body(uidx_hbm, midx_hbm, uR_hbm, mR_hbm, ug_hbm, mg_hbm,
                 idx_v, q_v, rows_v, sem):
    wid = lax.axis_index("s") * 2 + lax.axis_index("c")
    base = wid * B_PER_W
    for idx_hbm, R_hbm, out_hbm in ((uidx_hbm, uR_hbm, ug_hbm),
                                    (midx_hbm, mR_hbm, mg_hbm)):
        pltpu.sync_copy(idx_hbm.at[pl.ds(base, B_PER_W)], idx_v)
        for j in range(B_PER_W // 16):
            v = idx_v[pl.ds(16 * j, 16)]
            q_v[pl.ds(16 * j, 16)] = jnp.bitwise_and(v, CHUNK - 1)
        pltpu.async_copy(R_hbm.at[q_v], rows_v, sem).wait()
        pltpu.sync_copy(rows_v, out_hbm.at[pl.ds(base, B_PER_W)])


def _sc_gather(uidx, midx, uR, mR):
    mesh = plsc.VectorSubcoreMesh(core_axis_name="c", subcore_axis_name="s")
    return pl.kernel(
        _gather_body,
        mesh=mesh,
        out_type=[
            jax.ShapeDtypeStruct((BATCH, 128), jnp.float32),
            jax.ShapeDtypeStruct((BATCH, 128), jnp.float32),
        ],
        scratch_types=[
            pltpu.VMEM((B_PER_W,), jnp.int32),
            pltpu.VMEM((B_PER_W,), jnp.int32),
            pltpu.VMEM((B_PER_W, 128), jnp.float32),
            pltpu.SemaphoreType.DMA,
        ],
    )(uidx, midx, uR, mR)


def _mlp_body(ug_ref, mg_ref, uoh_ref, moh_ref, e_ref, w1a_ref, w1b_ref,
              b1_ref, w2_ref, b2_ref, o_ref):
    mu = jnp.dot(uoh_ref[...], e_ref[...], preferred_element_type=jnp.float32)
    mm = jnp.dot(moh_ref[...], e_ref[...], preferred_element_type=jnp.float32)
    h = (jnp.dot(ug_ref[...] * mu, w1a_ref[...],
                 preferred_element_type=jnp.float32)
         + jnp.dot(mg_ref[...] * mm, w1b_ref[...],
                   preferred_element_type=jnp.float32)
         + b1_ref[...])
    h = jnp.maximum(h, 0.0)
    o = jnp.dot(h, w2_ref[...], preferred_element_type=jnp.float32) + b2_ref[...]
    o_ref[...] = 5.0 * jax.nn.sigmoid(o)


def kernel(inputs, user_embedding, movie_embedding, W1, b1, W2, b2):
    uidx = inputs[:, 0]
    midx = inputs[:, 1]
    uR = _repack(user_embedding.T)
    mR = _repack(movie_embedding.T)
    ug, mg = _sc_gather(uidx, midx, uR, mR)

    uoh = jax.nn.one_hot(uidx // CHUNK, 4, dtype=jnp.float32)
    moh = jax.nn.one_hot(midx // CHUNK, 4, dtype=jnp.float32)
    expander = jnp.repeat(jnp.eye(4, dtype=jnp.float32), EMBED, axis=1)
    w1a_stack = jnp.tile(W1[:EMBED], (4, 1))
    w1b_stack = jnp.tile(W1[EMBED:], (4, 1))

    BT = 4096
    out = pl.pallas_call(
        _mlp_body,
        grid=(BATCH // BT,),
        in_specs=[
            pl.BlockSpec((BT, 128), lambda i: (i, 0)),
            pl.BlockSpec((BT, 128), lambda i: (i, 0)),
            pl.BlockSpec((BT, 4), lambda i: (i, 0)),
            pl.BlockSpec((BT, 4), lambda i: (i, 0)),
            pl.BlockSpec((4, 128), lambda i: (0, 0)),
            pl.BlockSpec((128, EMBED), lambda i: (0, 0)),
            pl.BlockSpec((128, EMBED), lambda i: (0, 0)),
            pl.BlockSpec((1, EMBED), lambda i: (0, 0)),
            pl.BlockSpec((EMBED, 1), lambda i: (0, 0)),
            pl.BlockSpec((1, 1), lambda i: (0, 0)),
        ],
        out_specs=pl.BlockSpec((BT, 1), lambda i: (i, 0)),
        out_shape=jax.ShapeDtypeStruct((BATCH, 1), jnp.float32),
    )(ug, mg, uoh, moh, expander, w1a_stack, w1b_stack,
      b1.reshape(1, EMBED), W2, b2.reshape(1, 1))
    return out.reshape(-1)


# speedup vs baseline: 12.3614x; 1.1723x over previous
import jax
import jax.numpy as jnp
from jax import lax
from jax.experimental import pallas as pl
from jax.experimental.pallas import tpu as pltpu
from jax.experimental.pallas import tpu_sc as plsc

EMBED = 32
BATCH = 16384
NUM_ROWS = 1000000
CHUNK = 1 << 17
OUT_BLK = 8192
GRID = CHUNK // OUT_BLK
COL_BLOCKS = (NUM_ROWS + OUT_BLK - 1) // OUT_BLK - 1
NUM_WORKERS = 32
B_PER_W = BATCH // NUM_WORKERS


def _bf16_hi_bits(y):
    yi = lax.bitcast_convert_type(y, jnp.uint32)
    return jnp.bitwise_and(yi + jnp.uint32(0x8000), jnp.uint32(0xFFFF0000))


def _repack_body(c0, c1, c2, c3, c4, c5, c6, c7, out_ref):
    x7 = c7[...]
    x7 = jnp.where(jnp.abs(x7) <= jnp.float32(3.0e38), x7, 0.0)
    lo = jnp.transpose(jnp.concatenate(
        [c0[...], c1[...], c2[...], c3[...]], axis=0))
    hi = jnp.transpose(jnp.concatenate(
        [c4[...], c5[...], c6[...], x7], axis=0))
    word = jnp.bitwise_or(
        _bf16_hi_bits(hi),
        jnp.right_shift(_bf16_hi_bits(lo), jnp.uint32(16)))
    out_ref[...] = lax.bitcast_convert_type(word, jnp.float32)


def _repack(tabT):
    specs = [
        pl.BlockSpec((32, OUT_BLK),
                     (lambda k, uu=u: (0, jnp.minimum(GRID * uu + k, COL_BLOCKS))))
        for u in range(8)
    ]
    return pl.pallas_call(
        _repack_body,
        grid=(GRID,),
        in_specs=specs,
        out_specs=pl.BlockSpec((OUT_BLK, 128), lambda k: (k, 0)),
        out_shape=jax.ShapeDtypeStruct((CHUNK, 128), jnp.float32),
    )(*([tabT] * 8))


def _gather_body(uidx_hbm, midx_hbm, uR_hbm, mR_hbm, ug_hbm, mg_hbm,
                 idx_v, q_v, rows_v, sem):
    wid = lax.axis_index("s") * 2 + lax.axis_index("c")
    base = wid * B_PER_W
    for idx_hbm, R_hbm, out_hbm in ((uidx_hbm, uR_hbm, ug_hbm),
                                    (midx_hbm, mR_hbm, mg_hbm)):
        pltpu.sync_copy(idx_hbm.at[pl.ds(base, B_PER_W)], idx_v)
        for j in range(B_PER_W // 16):
            v = idx_v[pl.ds(16 * j, 16)]
            q_v[pl.ds(16 * j, 16)] = jnp.bitwise_and(v, CHUNK - 1)
        pltpu.async_copy(R_hbm.at[q_v], rows_v, sem).wait()
        pltpu.sync_copy(rows_v, out_hbm.at[pl.ds(base, B_PER_W)])


def _sc_gather(uidx, midx, uR, mR):
    mesh = plsc.VectorSubcoreMesh(core_axis_name="c", subcore_axis_name="s")
    return pl.kernel(
        _gather_body,
        mesh=mesh,
        out_type=[
            jax.ShapeDtypeStruct((BATCH, 128), jnp.float32),
            jax.ShapeDtypeStruct((BATCH, 128), jnp.float32),
        ],
        scratch_types=[
            pltpu.VMEM((B_PER_W,), jnp.int32),
            pltpu.VMEM((B_PER_W,), jnp.int32),
            pltpu.VMEM((B_PER_W, 128), jnp.float32),
            pltpu.SemaphoreType.DMA,
        ],
    )(uidx, midx, uR, mR)


def _unpack_select(w_ref, mlo, mhi):
    wi = lax.bitcast_convert_type(w_ref[...], jnp.uint32)
    lo = lax.bitcast_convert_type(
        jnp.left_shift(wi, jnp.uint32(16)), jnp.float32)
    hi = lax.bitcast_convert_type(
        jnp.bitwise_and(wi, jnp.uint32(0xFFFF0000)), jnp.float32)
    return lo * mlo + hi * mhi


def _mlp_body(ug_ref, mg_ref, uoh_ref, moh_ref, elo_ref, ehi_ref,
              w1a_ref, w1b_ref, b1_ref, w2_ref, b2_ref, o_ref):
    elo = elo_ref[...]
    ehi = ehi_ref[...]
    u_lo = jnp.dot(uoh_ref[...], elo, preferred_element_type=jnp.float32)
    u_hi = jnp.dot(uoh_ref[...], ehi, preferred_element_type=jnp.float32)
    m_lo = jnp.dot(moh_ref[...], elo, preferred_element_type=jnp.float32)
    m_hi = jnp.dot(moh_ref[...], ehi, preferred_element_type=jnp.float32)
    xu = _unpack_select(ug_ref, u_lo, u_hi)
    xm = _unpack_select(mg_ref, m_lo, m_hi)
    h = (jnp.dot(xu, w1a_ref[...], preferred_element_type=jnp.float32)
         + jnp.dot(xm, w1b_ref[...], preferred_element_type=jnp.float32)
         + b1_ref[...])
    h = jnp.maximum(h, 0.0)
    o = jnp.dot(h, w2_ref[...], preferred_element_type=jnp.float32) + b2_ref[...]
    o_ref[...] = 5.0 * jax.nn.sigmoid(o)


def kernel(inputs, user_embedding, movie_embedding, W1, b1, W2, b2):
    uidx = inputs[:, 0]
    midx = inputs[:, 1]
    uR = _repack(user_embedding.T)
    mR = _repack(movie_embedding.T)
    ug, mg = _sc_gather(uidx, midx, uR, mR)

    uoh = jax.nn.one_hot(uidx // CHUNK, 8, dtype=jnp.float32)
    moh = jax.nn.one_hot(midx // CHUNK, 8, dtype=jnp.float32)
    win = jnp.repeat(jnp.eye(4, dtype=jnp.float32), EMBED, axis=1)
    zeros = jnp.zeros_like(win)
    e_lo = jnp.concatenate([win, zeros], axis=0)
    e_hi = jnp.concatenate([zeros, win], axis=0)
    w1a_stack = jnp.tile(W1[:EMBED], (4, 1))
    w1b_stack = jnp.tile(W1[EMBED:], (4, 1))

    BT = 4096
    out = pl.pallas_call(
        _mlp_body,
        grid=(BATCH // BT,),
        in_specs=[
            pl.BlockSpec((BT, 128), lambda i: (i, 0)),
            pl.BlockSpec((BT, 128), lambda i: (i, 0)),
            pl.BlockSpec((BT, 8), lambda i: (i, 0)),
            pl.BlockSpec((BT, 8), lambda i: (i, 0)),
            pl.BlockSpec((8, 128), lambda i: (0, 0)),
            pl.BlockSpec((8, 128), lambda i: (0, 0)),
            pl.BlockSpec((128, EMBED), lambda i: (0, 0)),
            pl.BlockSpec((128, EMBED), lambda i: (0, 0)),
            pl.BlockSpec((1, EMBED), lambda i: (0, 0)),
            pl.BlockSpec((EMBED, 1), lambda i: (0, 0)),
            pl.BlockSpec((1, 1), lambda i: (0, 0)),
        ],
        out_specs=pl.BlockSpec((BT, 1), lambda i: (i, 0)),
        out_shape=jax.ShapeDtypeStruct((BATCH, 1), jnp.float32),
    )(ug, mg, uoh, moh, e_lo, e_hi, w1a_stack, w1b_stack,
      b1.reshape(1, EMBED), W2, b2.reshape(1, 1))
    return out.reshape(-1)


# speedup vs baseline: 12.6645x; 1.0245x over previous
import jax
import jax.numpy as jnp
from jax import lax
from jax.experimental import pallas as pl
from jax.experimental.pallas import tpu as pltpu
from jax.experimental.pallas import tpu_sc as plsc

EMBED = 32
BATCH = 16384
NUM_ROWS = 1000000
CHUNK = 1 << 17
OUT_BLK = 16384
GRID = CHUNK // OUT_BLK
COL_BLOCKS = (NUM_ROWS + OUT_BLK - 1) // OUT_BLK - 1
NUM_WORKERS = 32
B_PER_W = BATCH // NUM_WORKERS


def _bf16_hi_bits(y):
    yi = lax.bitcast_convert_type(y, jnp.uint32)
    return jnp.bitwise_and(yi + jnp.uint32(0x8000), jnp.uint32(0xFFFF0000))


def _repack_body(c0, c1, c2, c3, c4, c5, c6, c7, out_ref):
    x7 = c7[...]
    x7 = jnp.where(jnp.abs(x7) <= jnp.float32(3.0e38), x7, 0.0)
    lo = jnp.transpose(jnp.concatenate(
        [c0[...], c1[...], c2[...], c3[...]], axis=0))
    hi = jnp.transpose(jnp.concatenate(
        [c4[...], c5[...], c6[...], x7], axis=0))
    word = jnp.bitwise_or(
        _bf16_hi_bits(hi),
        jnp.right_shift(_bf16_hi_bits(lo), jnp.uint32(16)))
    out_ref[...] = lax.bitcast_convert_type(word, jnp.float32)


def _repack(tabT):
    specs = [
        pl.BlockSpec((32, OUT_BLK),
                     (lambda k, uu=u: (0, jnp.minimum(GRID * uu + k, COL_BLOCKS))))
        for u in range(8)
    ]
    return pl.pallas_call(
        _repack_body,
        grid=(GRID,),
        in_specs=specs,
        out_specs=pl.BlockSpec((OUT_BLK, 128), lambda k: (k, 0)),
        out_shape=jax.ShapeDtypeStruct((CHUNK, 128), jnp.float32),
    )(*([tabT] * 8))


def _gather_body(uidx_hbm, midx_hbm, uR_hbm, mR_hbm, ug_hbm, mg_hbm,
                 idx_v, q_v, rows_v, sem):
    wid = lax.axis_index("s") * 2 + lax.axis_index("c")
    base = wid * B_PER_W
    for idx_hbm, R_hbm, out_hbm in ((uidx_hbm, uR_hbm, ug_hbm),
                                    (midx_hbm, mR_hbm, mg_hbm)):
        pltpu.sync_copy(idx_hbm.at[pl.ds(base, B_PER_W)], idx_v)
        for j in range(B_PER_W // 16):
            v = idx_v[pl.ds(16 * j, 16)]
            q_v[pl.ds(16 * j, 16)] = jnp.bitwise_and(v, CHUNK - 1)
        pltpu.async_copy(R_hbm.at[q_v], rows_v, sem).wait()
        pltpu.sync_copy(rows_v, out_hbm.at[pl.ds(base, B_PER_W)])


def _sc_gather(uidx, midx, uR, mR):
    mesh = plsc.VectorSubcoreMesh(core_axis_name="c", subcore_axis_name="s")
    return pl.kernel(
        _gather_body,
        mesh=mesh,
        out_type=[
            jax.ShapeDtypeStruct((BATCH, 128), jnp.float32),
            jax.ShapeDtypeStruct((BATCH, 128), jnp.float32),
        ],
        scratch_types=[
            pltpu.VMEM((B_PER_W,), jnp.int32),
            pltpu.VMEM((B_PER_W,), jnp.int32),
            pltpu.VMEM((B_PER_W, 128), jnp.float32),
            pltpu.SemaphoreType.DMA,
        ],
    )(uidx, midx, uR, mR)


def _unpack_select(w_ref, mlo, mhi):
    wi = lax.bitcast_convert_type(w_ref[...], jnp.uint32)
    lo = lax.bitcast_convert_type(
        jnp.left_shift(wi, jnp.uint32(16)), jnp.float32)
    hi = lax.bitcast_convert_type(
        jnp.bitwise_and(wi, jnp.uint32(0xFFFF0000)), jnp.float32)
    return lo * mlo + hi * mhi


def _mlp_body(ug_ref, mg_ref, uoh_ref, moh_ref, elo_ref, ehi_ref,
              w1a_ref, w1b_ref, b1_ref, w2_ref, b2_ref, o_ref):
    elo = elo_ref[...]
    ehi = ehi_ref[...]
    u_lo = jnp.dot(uoh_ref[...], elo, preferred_element_type=jnp.float32)
    u_hi = jnp.dot(uoh_ref[...], ehi, preferred_element_type=jnp.float32)
    m_lo = jnp.dot(moh_ref[...], elo, preferred_element_type=jnp.float32)
    m_hi = jnp.dot(moh_ref[...], ehi, preferred_element_type=jnp.float32)
    xu = _unpack_select(ug_ref, u_lo, u_hi)
    xm = _unpack_select(mg_ref, m_lo, m_hi)
    h = (jnp.dot(xu, w1a_ref[...], preferred_element_type=jnp.float32)
         + jnp.dot(xm, w1b_ref[...], preferred_element_type=jnp.float32)
         + b1_ref[...])
    h = jnp.maximum(h, 0.0)
    o = jnp.dot(h, w2_ref[...], preferred_element_type=jnp.float32) + b2_ref[...]
    o_ref[...] = 5.0 * jax.nn.sigmoid(o)


def kernel(inputs, user_embedding, movie_embedding, W1, b1, W2, b2):
    uidx = inputs[:, 0]
    midx = inputs[:, 1]
    uR = _repack(user_embedding.T)
    mR = _repack(movie_embedding.T)
    ug, mg = _sc_gather(uidx, midx, uR, mR)

    uoh = jax.nn.one_hot(uidx // CHUNK, 8, dtype=jnp.float32)
    moh = jax.nn.one_hot(midx // CHUNK, 8, dtype=jnp.float32)
    win = jnp.repeat(jnp.eye(4, dtype=jnp.float32), EMBED, axis=1)
    zeros = jnp.zeros_like(win)
    e_lo = jnp.concatenate([win, zeros], axis=0)
    e_hi = jnp.concatenate([zeros, win], axis=0)
    w1a_stack = jnp.tile(W1[:EMBED], (4, 1))
    w1b_stack = jnp.tile(W1[EMBED:], (4, 1))

    BT = 4096
    out = pl.pallas_call(
        _mlp_body,
        grid=(BATCH // BT,),
        in_specs=[
            pl.BlockSpec((BT, 128), lambda i: (i, 0)),
            pl.BlockSpec((BT, 128), lambda i: (i, 0)),
            pl.BlockSpec((BT, 8), lambda i: (i, 0)),
            pl.BlockSpec((BT, 8), lambda i: (i, 0)),
            pl.BlockSpec((8, 128), lambda i: (0, 0)),
            pl.BlockSpec((8, 128), lambda i: (0, 0)),
            pl.BlockSpec((128, EMBED), lambda i: (0, 0)),
            pl.BlockSpec((128, EMBED), lambda i: (0, 0)),
            pl.BlockSpec((1, EMBED), lambda i: (0, 0)),
            pl.BlockSpec((EMBED, 1), lambda i: (0, 0)),
            pl.BlockSpec((1, 1), lambda i: (0, 0)),
        ],
        out_specs=pl.BlockSpec((BT, 1), lambda i: (i, 0)),
        out_shape=jax.ShapeDtypeStruct((BATCH, 1), jnp.float32),
    )(ug, mg, uoh, moh, e_lo, e_hi, w1a_stack, w1b_stack,
      b1.reshape(1, EMBED), W2, b2.reshape(1, 1))
    return out.reshape(-1)


# speedup vs baseline: 12.6756x; 1.0009x over previous
import jax
import jax.numpy as jnp
from jax import lax
from jax.experimental import pallas as pl
from jax.experimental.pallas import tpu as pltpu
from jax.experimental.pallas import tpu_sc as plsc

EMBED = 32
BATCH = 16384
NUM_ROWS = 1000000
CHUNK = 1 << 17
OUT_BLK = 16384
GRID = CHUNK // OUT_BLK
COL_BLOCKS = (NUM_ROWS + OUT_BLK - 1) // OUT_BLK - 1
NUM_WORKERS = 32
B_PER_W = BATCH // NUM_WORKERS


def _bf16_hi_bits(y):
    yi = lax.bitcast_convert_type(y, jnp.uint32)
    return jnp.bitwise_and(yi + jnp.uint32(0x8000), jnp.uint32(0xFFFF0000))


def _repack_body(c0, c1, c2, c3, c4, c5, c6, c7, out_ref):
    x7 = c7[...]
    x7 = jnp.where(jnp.abs(x7) <= jnp.float32(3.0e38), x7, 0.0)
    lo = jnp.transpose(jnp.concatenate(
        [c0[...], c1[...], c2[...], c3[...]], axis=0))
    hi = jnp.transpose(jnp.concatenate(
        [c4[...], c5[...], c6[...], x7], axis=0))
    word = jnp.bitwise_or(
        _bf16_hi_bits(hi),
        jnp.right_shift(_bf16_hi_bits(lo), jnp.uint32(16)))
    out_ref[...] = lax.bitcast_convert_type(word, jnp.float32)


def _repack(tabT):
    specs = [
        pl.BlockSpec((32, OUT_BLK),
                     (lambda k, uu=u: (0, jnp.minimum(GRID * uu + k, COL_BLOCKS))))
        for u in range(8)
    ]
    return pl.pallas_call(
        _repack_body,
        grid=(GRID,),
        in_specs=specs,
        out_specs=pl.BlockSpec((OUT_BLK, 128), lambda k: (k, 0)),
        out_shape=jax.ShapeDtypeStruct((CHUNK, 128), jnp.float32),
    )(*([tabT] * 8))


def _gather_body(idx_hbm, R_hbm, out_hbm, idx_v, q_v, rows_v, sem):
    wid = lax.axis_index("s") * 2 + lax.axis_index("c")
    base = wid * B_PER_W
    pltpu.sync_copy(idx_hbm.at[pl.ds(base, B_PER_W)], idx_v)
    for j in range(B_PER_W // 16):
        v = idx_v[pl.ds(16 * j, 16)]
        q_v[pl.ds(16 * j, 16)] = jnp.bitwise_and(v, CHUNK - 1)
    pltpu.async_copy(R_hbm.at[q_v], rows_v, sem).wait()
    pltpu.sync_copy(rows_v, out_hbm.at[pl.ds(base, B_PER_W)])


def _sc_gather(idx, R):
    mesh = plsc.VectorSubcoreMesh(core_axis_name="c", subcore_axis_name="s")
    return pl.kernel(
        _gather_body,
        mesh=mesh,
        out_type=jax.ShapeDtypeStruct((BATCH, 128), jnp.float32),
        scratch_types=[
            pltpu.VMEM((B_PER_W,), jnp.int32),
            pltpu.VMEM((B_PER_W,), jnp.int32),
            pltpu.VMEM((B_PER_W, 128), jnp.float32),
            pltpu.SemaphoreType.DMA,
        ],
    )(idx, R)


def _unpack_select(w_ref, mlo, mhi):
    wi = lax.bitcast_convert_type(w_ref[...], jnp.uint32)
    lo = lax.bitcast_convert_type(
        jnp.left_shift(wi, jnp.uint32(16)), jnp.float32)
    hi = lax.bitcast_convert_type(
        jnp.bitwise_and(wi, jnp.uint32(0xFFFF0000)), jnp.float32)
    return lo * mlo + hi * mhi


def _mlp_body(ug_ref, mg_ref, uoh_ref, moh_ref, elo_ref, ehi_ref,
              w1a_ref, w1b_ref, b1_ref, w2_ref, b2_ref, o_ref):
    elo = elo_ref[...]
    ehi = ehi_ref[...]
    u_lo = jnp.dot(uoh_ref[...], elo, preferred_element_type=jnp.float32)
    u_hi = jnp.dot(uoh_ref[...], ehi, preferred_element_type=jnp.float32)
    m_lo = jnp.dot(moh_ref[...], elo, preferred_element_type=jnp.float32)
    m_hi = jnp.dot(moh_ref[...], ehi, preferred_element_type=jnp.float32)
    xu = _unpack_select(ug_ref, u_lo, u_hi)
    xm = _unpack_select(mg_ref, m_lo, m_hi)
    h = (jnp.dot(xu, w1a_ref[...], preferred_element_type=jnp.float32)
         + jnp.dot(xm, w1b_ref[...], preferred_element_type=jnp.float32)
         + b1_ref[...])
    h = jnp.maximum(h, 0.0)
    o = jnp.dot(h, w2_ref[...], preferred_element_type=jnp.float32) + b2_ref[...]
    o_ref[...] = 5.0 * jax.nn.sigmoid(o)


def kernel(inputs, user_embedding, movie_embedding, W1, b1, W2, b2):
    uidx = inputs[:, 0]
    midx = inputs[:, 1]
    uR = _repack(user_embedding.T)
    ug = _sc_gather(uidx, uR)
    mR = _repack(movie_embedding.T)
    mg = _sc_gather(midx, mR)

    uoh = jax.nn.one_hot(uidx // CHUNK, 8, dtype=jnp.float32)
    moh = jax.nn.one_hot(midx // CHUNK, 8, dtype=jnp.float32)
    win = jnp.repeat(jnp.eye(4, dtype=jnp.float32), EMBED, axis=1)
    zeros = jnp.zeros_like(win)
    e_lo = jnp.concatenate([win, zeros], axis=0)
    e_hi = jnp.concatenate([zeros, win], axis=0)
    w1a_stack = jnp.tile(W1[:EMBED], (4, 1))
    w1b_stack = jnp.tile(W1[EMBED:], (4, 1))

    BT = 4096
    out = pl.pallas_call(
        _mlp_body,
        grid=(BATCH // BT,),
        in_specs=[
            pl.BlockSpec((BT, 128), lambda i: (i, 0)),
            pl.BlockSpec((BT, 128), lambda i: (i, 0)),
            pl.BlockSpec((BT, 8), lambda i: (i, 0)),
            pl.BlockSpec((BT, 8), lambda i: (i, 0)),
            pl.BlockSpec((8, 128), lambda i: (0, 0)),
            pl.BlockSpec((8, 128), lambda i: (0, 0)),
            pl.BlockSpec((128, EMBED), lambda i: (0, 0)),
            pl.BlockSpec((128, EMBED), lambda i: (0, 0)),
            pl.BlockSpec((1, EMBED), lambda i: (0, 0)),
            pl.BlockSpec((EMBED, 1), lambda i: (0, 0)),
            pl.BlockSpec((1, 1), lambda i: (0, 0)),
        ],
        out_specs=pl.BlockSpec((BT, 1), lambda i: (i, 0)),
        out_shape=jax.ShapeDtypeStruct((BATCH, 1), jnp.float32),
    )(ug, mg, uoh, moh, e_lo, e_hi, w1a_stack, w1b_stack,
      b1.reshape(1, EMBED), W2, b2.reshape(1, 1))
    return out.reshape(-1)
